# Initial kernel scaffold; baseline (speedup 1.0000x reference)
import jax
import jax.numpy as jnp
from jax.experimental import pallas as pl


def kernel(u_emb, i_emb, edge_index, pVui, pKiu, graph, last_u, last_i, W1, W2, W1b, W2b, W3, W4):
    raise NotImplementedError("write your pallas kernel here")


# speedup vs baseline: 3.4498x; 3.4498x over previous
import jax
import jax.numpy as jnp
from jax.experimental import pallas as pl
from jax.experimental.pallas import tpu as pltpu

_CP = pltpu.CompilerParams(vmem_limit_bytes=100 * 1024 * 1024)

U = 4096
I = 4096
D = 256
E = 131072
EB = 512
RB = 512
SQRT_D = 16.0


def _dotT(x, w):
    return jax.lax.dot_general(x, w, (((1,), (1,)), ((), ())),
                               preferred_element_type=jnp.float32)


def _dotA(x, y):
    return jax.lax.dot_general(x, y, (((1,), (0,)), ((), ())),
                               preferred_element_type=jnp.float32)


def _dotTA(x, y):
    return jax.lax.dot_general(x, y, (((0,), (0,)), ((), ())),
                               preferred_element_type=jnp.float32)


def _prologue_kernel(u_blk, i_blk, u_full, i_full, lui, lii,
                     W1, W2, W1b, W2b, W3, W4, catu, cati):
    um_att = _dotT(u_blk[...], W2[...])
    um_b = _dotT(u_blk[...], W2b[...])
    im_att = _dotT(i_blk[...], W1[...])
    im_b = _dotT(i_blk[...], W1b[...])
    oh_li = (lui[...] == jax.lax.broadcasted_iota(jnp.int32, (RB, I), 1)
             ).astype(jnp.float32)
    li = _dotT(_dotA(oh_li, i_full[...]), W3[...])
    oh_lu = (lii[...] == jax.lax.broadcasted_iota(jnp.int32, (RB, U), 1)
             ).astype(jnp.float32)
    lu = _dotT(_dotA(oh_lu, u_full[...]), W4[...])
    catu[...] = jnp.concatenate([um_att, um_b, li, lu], axis=1)
    cati[...] = jnp.concatenate([im_att, im_b], axis=1)


def _passA_kernel(ui, ii, pV, pK, catu, cati,
                  denU, denI, exs, g1, g2, g3, g4):
    P = (ui[...] == jax.lax.broadcasted_iota(jnp.int32, (EB, U), 1)
         ).astype(jnp.float32)
    Q = (ii[...] == jax.lax.broadcasted_iota(jnp.int32, (EB, I), 1)
         ).astype(jnp.float32)
    GU = _dotA(P, catu[...])
    GI = _dotA(Q, cati[...])
    A = GU[:, :D]
    umb_g = GU[:, D:2 * D]
    Li = GU[:, 2 * D:3 * D]
    Lu = GU[:, 3 * D:]
    B = GI[:, :D]
    imb_g = GI[:, D:]
    pv = pV[...]
    pk = pK[...]
    ev = jnp.sum(A * B, axis=1, keepdims=True)
    ex1 = jnp.exp((ev + jnp.sum(A * pv, axis=1, keepdims=True)) / SQRT_D)
    ex2 = jnp.exp((ev + jnp.sum(B * pk, axis=1, keepdims=True)) / SQRT_D)
    ex3 = jnp.exp(jnp.sum(Li * B, axis=1, keepdims=True) / SQRT_D)
    ex4 = jnp.exp(jnp.sum(Lu * B, axis=1, keepdims=True) / SQRT_D)
    zpad = jnp.zeros((EB, 126), dtype=jnp.float32)
    EXu = jnp.concatenate([ex1, ex3, zpad], axis=1)
    EXi = jnp.concatenate([ex2, ex4, zpad], axis=1)

    @pl.when(pl.program_id(0) == 0)
    def _():
        denU[...] = jnp.zeros_like(denU)
        denI[...] = jnp.zeros_like(denI)

    denU[...] += _dotTA(P, EXu)
    denI[...] += _dotTA(Q, EXi)
    exs[...] = jnp.concatenate([ex1, ex2, ex3, ex4], axis=1)
    g1[...] = imb_g + pk
    g2[...] = B + 1.0
    g3[...] = umb_g + pv
    g4[...] = A + 1.0


def _passB_kernel(ui, ii, exs, g1, g2, g3, g4, denU, denI, outU, outI):
    P = (ui[...] == jax.lax.broadcasted_iota(jnp.int32, (EB, U), 1)
         ).astype(jnp.float32)
    Q = (ii[...] == jax.lax.broadcasted_iota(jnp.int32, (EB, I), 1)
         ).astype(jnp.float32)
    dU = _dotA(P, denU[...])
    dI = _dotA(Q, denI[...])
    ex = exs[...]
    alpha = ex[:, 0:1] / (dU[:, 0:1] + 1e-16)
    beta = ex[:, 1:2] / (dI[:, 0:1] + 1e-16)
    alps = ex[:, 2:3] / (dU[:, 1:2] + 1e-16)
    bets = ex[:, 3:4] / (dI[:, 1:2] + 1e-16)
    XU = jnp.concatenate([alpha * g1[...], alps * g2[...]], axis=1)
    XI = jnp.concatenate([beta * g3[...], bets * g4[...]], axis=1)

    @pl.when(pl.program_id(0) == 0)
    def _():
        outU[...] = jnp.zeros_like(outU)
        outI[...] = jnp.zeros_like(outI)

    outU[...] += _dotTA(P, XU)
    outI[...] += _dotTA(Q, XI)


def kernel(u_emb, i_emb, edge_index, pVui, pKiu, graph, last_u, last_i,
           W1, W2, W1b, W2b, W3, W4):
    f32 = jnp.float32
    uiT = edge_index[0].reshape(E, 1)
    iiT = edge_index[1].reshape(E, 1)
    luiT = last_u[1].reshape(U, 1)
    liiT = last_i[1].reshape(U, 1)

    wspec = pl.BlockSpec((D, D), lambda g: (0, 0))
    full_u = pl.BlockSpec((U, D), lambda g: (0, 0))
    full_i = pl.BlockSpec((I, D), lambda g: (0, 0))

    catu, cati = pl.pallas_call(
        _prologue_kernel,
        grid=(U // RB,),
        compiler_params=_CP,
        in_specs=[
            pl.BlockSpec((RB, D), lambda g: (g, 0)),
            pl.BlockSpec((RB, D), lambda g: (g, 0)),
            full_u, full_i,
            pl.BlockSpec((RB, 1), lambda g: (g, 0)),
            pl.BlockSpec((RB, 1), lambda g: (g, 0)),
            wspec, wspec, wspec, wspec, wspec, wspec,
        ],
        out_specs=[
            pl.BlockSpec((RB, 4 * D), lambda g: (g, 0)),
            pl.BlockSpec((RB, 2 * D), lambda g: (g, 0)),
        ],
        out_shape=[
            jax.ShapeDtypeStruct((U, 4 * D), f32),
            jax.ShapeDtypeStruct((I, 2 * D), f32),
        ],
    )(u_emb, i_emb, u_emb, i_emb, luiT, liiT, W1, W2, W1b, W2b, W3, W4)

    nblk = E // EB
    eidx = pl.BlockSpec((EB, 1), lambda g: (g, 0))
    evec = pl.BlockSpec((EB, D), lambda g: (g, 0))
    acc128 = pl.BlockSpec((U, 128), lambda g: (0, 0))

    denU, denI, exs, g1, g2, g3, g4 = pl.pallas_call(
        _passA_kernel,
        grid=(nblk,),
        compiler_params=_CP,
        in_specs=[
            eidx, eidx, evec, evec,
            pl.BlockSpec((U, 4 * D), lambda g: (0, 0)),
            pl.BlockSpec((I, 2 * D), lambda g: (0, 0)),
        ],
        out_specs=[
            acc128, acc128,
            pl.BlockSpec((EB, 4), lambda g: (g, 0)),
            evec, evec, evec, evec,
        ],
        out_shape=[
            jax.ShapeDtypeStruct((U, 128), f32),
            jax.ShapeDtypeStruct((I, 128), f32),
            jax.ShapeDtypeStruct((E, 4), f32),
            jax.ShapeDtypeStruct((E, D), f32),
            jax.ShapeDtypeStruct((E, D), f32),
            jax.ShapeDtypeStruct((E, D), f32),
            jax.ShapeDtypeStruct((E, D), f32),
        ],
    )(uiT, iiT, pVui, pKiu, catu, cati)

    outU, outI = pl.pallas_call(
        _passB_kernel,
        grid=(nblk,),
        compiler_params=_CP,
        in_specs=[
            eidx, eidx,
            pl.BlockSpec((EB, 4), lambda g: (g, 0)),
            evec, evec, evec, evec,
            acc128, acc128,
        ],
        out_specs=[
            pl.BlockSpec((U, 2 * D), lambda g: (0, 0)),
            pl.BlockSpec((I, 2 * D), lambda g: (0, 0)),
        ],
        out_shape=[
            jax.ShapeDtypeStruct((U, 2 * D), f32),
            jax.ShapeDtypeStruct((I, 2 * D), f32),
        ],
    )(uiT, iiT, exs, g1, g2, g3, g4, denU, denI)

    hLu = outU[:, :D]
    hSu = outU[:, D:]
    hLi = outI[:, :D]
    hSi = outI[:, D:]
    return (hLu, hSu, hLi, hSi)
